# baseline (speedup 1.0000x reference)
import functools

import jax
import jax.numpy as jnp
from jax import lax
from jax.experimental import pallas as pl
from jax.experimental.pallas import tpu as pltpu
from jax.experimental.pallas import tpu_sc as plsc

_B, _D, _C, _K = 32, 128, 1000000, 64
_S = 16384
_NBLK = -(-_C // _S)
_CP = _NBLK * _S
_CHUNK = 128
_NCHUNK = _CP // _CHUNK
_NEG = -3.0e38
_BIGI = 0x7FFFFFFF
_GCH = 64
_NFULLG = _C // (_GCH * _CHUNK)
_NCAP = 4096
_PCAP = 4096
_NB = 128
_NC, _NS = 2, 16


def _phase_a_body(q_ref, k_ref, scores_ref, cmax_ref):
    i = pl.program_id(0)
    q = q_ref[...]
    k = k_ref[...]
    s = jax.lax.dot_general(
        q, k, (((1,), (1,)), ((), ())),
        preferred_element_type=jnp.float32)
    col = i * _S + jax.lax.broadcasted_iota(jnp.int32, (_B, _S), 1)
    s = jnp.where(col < _C, s, _NEG)
    s3 = s.reshape(_B, _S // _CHUNK, _CHUNK)
    scores_ref[...] = s3
    cmax_ref[...] = jnp.max(s3, axis=2)


_phase_a = pl.pallas_call(
    _phase_a_body,
    grid=(_NBLK,),
    in_specs=[
        pl.BlockSpec((_B, _D), lambda i: (0, 0)),
        pl.BlockSpec((_S, _D), lambda i: (i, 0)),
    ],
    out_specs=[
        pl.BlockSpec((_B, _S // _CHUNK, _CHUNK), lambda i: (0, i, 0)),
        pl.BlockSpec((_B, _S // _CHUNK), lambda i: (0, i)),
    ],
    out_shape=[
        jax.ShapeDtypeStruct((_B, _NCHUNK, _CHUNK), jnp.float32),
        jax.ShapeDtypeStruct((_B, _NCHUNK), jnp.float32),
    ],
    compiler_params=pltpu.CompilerParams(
        dimension_semantics=("arbitrary",),
    ),
)


@functools.partial(
    pl.kernel,
    mesh=plsc.VectorSubcoreMesh(core_axis_name="c", subcore_axis_name="s"),
    out_type=jax.ShapeDtypeStruct((_B, _K, _D), jnp.float32),
    scratch_types=[
        pltpu.VMEM((_NCHUNK,), jnp.float32),
        pltpu.VMEM((_NCAP + 128,), jnp.int32),
        pltpu.VMEM((_NB, _CHUNK), jnp.float32),
        pltpu.VMEM((_PCAP + 128,), jnp.float32),
        pltpu.VMEM((_PCAP + 128,), jnp.int32),
        pltpu.VMEM((_K,), jnp.int32),
        pltpu.VMEM((_K, _D), jnp.float32),
        pltpu.VMEM((16,), jnp.int32),
        pltpu.VMEM((16,), jnp.float32),
        pltpu.VMEM((16,), jnp.int32),
        pltpu.VMEM((_NCAP + 128,), jnp.float32),
        pltpu.VMEM((_NCAP + 128,), jnp.float32),
        pltpu.VMEM((_NCAP + 128,), jnp.int32),
        pltpu.VMEM((16,), jnp.float32),
        pltpu.SemaphoreType.DMA,
    ],
)
def _phase_b(scores2d_hbm, cmax_hbm, v_hbm, out_hbm,
             cmax_v, ids_v, buf_v, cv_v, ci_v, ti_v, vrows_v,
             pend_v, pcv_v, pci_v, cm_v, cm2_v, ids2_v, pcm_v, sem):
    r = lax.axis_index("s") * _NC + lax.axis_index("c")
    row_base = r * _NCHUNK
    lane = lax.iota(jnp.int32, 16)

    _dnums = lax.GatherDimensionNumbers(
        offset_dims=(), collapsed_slice_dims=(0,), start_index_map=(0,))

    def _lane_shuf(x, k):
        return lax.gather(x, (lane ^ k)[:, None], _dnums, slice_sizes=(1,),
                          mode=lax.GatherScatterMode.PROMISE_IN_BOUNDS)

    def _lanemax(x):
        for k in (8, 4, 2, 1):
            x = jnp.maximum(x, _lane_shuf(x, k))
        return x[0]

    def _lanemin(x):
        for k in (8, 4, 2, 1):
            x = jnp.minimum(x, _lane_shuf(x, k))
        return x[0]

    pltpu.sync_copy(cmax_hbm.at[r], cmax_v)

    def _grp(g, tau):
        m = cmax_v[pl.ds(g * _GCH, 16)]
        for j in range(1, _GCH // 16):
            m = jnp.maximum(m, cmax_v[pl.ds(g * _GCH + j * 16, 16)])
        return jnp.minimum(tau, _lanemax(m))
    tau = lax.fori_loop(0, _NFULLG, _grp, jnp.float32(3.0e38))

    def _fill_ids(i, c):
        ids_v[pl.ds(i * 16, 16)] = row_base + i * 16 + lane
        return c
    lax.fori_loop(0, (_NCAP + 128) // 16, _fill_ids, 0)

    def _fill_cand(i, c):
        cv_v[pl.ds(i * 16, 16)] = jnp.full((16,), _NEG, jnp.float32)
        ci_v[pl.ds(i * 16, 16)] = jnp.full((16,), _BIGI, jnp.int32)
        return c
    lax.fori_loop(0, (_PCAP + 128) // 16, _fill_cand, 0)

    safe_ids = row_base + lane
    pend_v[pl.ds(0, 16)] = safe_ids
    pcm_v[pl.ds(0, 16)] = jnp.full((16,), _NEG, jnp.float32)

    def _sel(i, off):
        def _ins(off):
            v = cmax_v[pl.ds(i * 16, 16)]
            pend = pend_v[pl.ds(0, 16)]
            pcm = pcm_v[pl.ds(0, 16)]
            for l in range(16):
                sc = v[l]
                ok = (sc >= tau) & (off < _NCAP)
                tgt = jnp.where(ok, lax.rem(off, 16), jnp.int32(16))
                pend = jnp.where(lane == tgt, row_base + i * 16 + l, pend)
                pcm = jnp.where(lane == tgt, sc, pcm)
                noff = off + jnp.where(ok, jnp.int32(1), jnp.int32(0))

                @pl.when(ok & (lax.rem(noff, 16) == 0))
                def _():
                    ids_v[pl.ds(noff - 16, 16)] = pend
                    cm_v[pl.ds(noff - 16, 16)] = pcm
                off = noff
            pend_v[pl.ds(0, 16)] = pend
            pcm_v[pl.ds(0, 16)] = pcm
            return off

        v = cmax_v[pl.ds(i * 16, 16)]
        return lax.cond(_lanemax(v) >= tau, _ins, lambda o: o, off)

    n_chunks = lax.fori_loop(0, _NCHUNK // 16, _sel, jnp.int32(0))

    @pl.when(lax.rem(n_chunks, 16) != 0)
    def _():
        ids_v[pl.ds((n_chunks // 16) * 16, 16)] = pend_v[pl.ds(0, 16)]
        cm_v[pl.ds((n_chunks // 16) * 16, 16)] = pcm_v[pl.ds(0, 16)]

    nv1 = (n_chunks + 15) // 16

    def _pad1(i, c):
        cm_v[pl.ds(n_chunks + i * 16, 16)] = jnp.full((16,), _NEG, jnp.float32)
        return c
    lax.fori_loop(0, 4, _pad1, 0)

    def _copy1(i, c):
        cm2_v[pl.ds(i * 16, 16)] = cm_v[pl.ds(i * 16, 16)]
        return c
    lax.fori_loop(0, nv1 + 1, _copy1, 0)

    nv1_4 = (nv1 + 3) // 4

    def _t2round(t, last):
        def _mx(i, m):
            for u in range(4):
                m = jnp.maximum(m, cm_v[pl.ds((i * 4 + u) * 16, 16)])
            return m
        mvec = lax.fori_loop(0, nv1_4, _mx, jnp.full((16,), _NEG, jnp.float32))
        mx = _lanemax(mvec)

        def _mi(i, mi):
            for u in range(4):
                hit = cm_v[pl.ds((i * 4 + u) * 16, 16)] == mx
                mi = jnp.minimum(mi, jnp.where(
                    hit, ids_v[pl.ds((i * 4 + u) * 16, 16)], jnp.int32(_BIGI)))
            return mi
        ivec = lax.fori_loop(0, nv1_4, _mi, jnp.full((16,), _BIGI, jnp.int32))
        ix = _lanemin(ivec)

        def _clr(i, c):
            for u in range(4):
                vv = cm_v[pl.ds((i * 4 + u) * 16, 16)]
                cc = ids_v[pl.ds((i * 4 + u) * 16, 16)]
                cm_v[pl.ds((i * 4 + u) * 16, 16)] = jnp.where(
                    vv == mx, jnp.where(cc == ix, jnp.float32(_NEG), vv), vv)
            return c
        lax.fori_loop(0, nv1_4, _clr, 0)
        return mx

    tau2 = lax.fori_loop(0, _K, _t2round, jnp.float32(_NEG))
    tauv = jnp.maximum(tau, tau2)

    def _fill_ids2(i, c):
        ids2_v[pl.ds(i * 16, 16)] = row_base + i * 16 + lane
        return c
    lax.fori_loop(0, (_NCAP + 128) // 16, _fill_ids2, 0)
    pend_v[pl.ds(0, 16)] = safe_ids

    def _sel2(i, off):
        def _ins(off):
            v = cm2_v[pl.ds(i * 16, 16)]
            idv = ids_v[pl.ds(i * 16, 16)]
            pend = pend_v[pl.ds(0, 16)]
            for l in range(16):
                sc = v[l]
                ok = (sc >= tauv) & (off < _NCAP)
                tgt = jnp.where(ok, lax.rem(off, 16), jnp.int32(16))
                pend = jnp.where(lane == tgt, idv[l], pend)
                noff = off + jnp.where(ok, jnp.int32(1), jnp.int32(0))

                @pl.when(ok & (lax.rem(noff, 16) == 0))
                def _():
                    ids2_v[pl.ds(noff - 16, 16)] = pend
                off = noff
            pend_v[pl.ds(0, 16)] = pend
            return off

        v = cm2_v[pl.ds(i * 16, 16)]
        return lax.cond(_lanemax(v) >= tauv, _ins, lambda o: o, off)

    n_chunks2 = lax.fori_loop(0, nv1, _sel2, jnp.int32(0))

    @pl.when(lax.rem(n_chunks2, 16) != 0)
    def _():
        ids2_v[pl.ds((n_chunks2 // 16) * 16, 16)] = pend_v[pl.ds(0, 16)]

    n_batch = (n_chunks + _NB - 1) // _NB

    pcv_v[pl.ds(0, 16)] = jnp.full((16,), _NEG, jnp.float32)
    pci_v[pl.ds(0, 16)] = safe_ids

    def _batch(b, off2):
        pltpu.async_copy(
            scores2d_hbm.at[ids_v.at[pl.ds(b * _NB, _NB)]], buf_v, sem,
        ).wait()
        jlim = jnp.minimum(n_chunks - b * _NB, _NB)

        def _chunk(j, off2):
            abs_id = ids_v[pl.ds(b * _NB + j, 16)][0]
            col0 = (abs_id - row_base) * _CHUNK

            def _vstep(v, off2):
                val = buf_v[j, pl.ds(v * 16, 16)]

                def _ins(off2):
                    pv = pcv_v[pl.ds(0, 16)]
                    pi = pci_v[pl.ds(0, 16)]
                    for l in range(16):
                        sc = val[l]
                        ok = (sc >= tau) & (off2 < _PCAP)
                        tgt = jnp.where(ok, lax.rem(off2, 16), jnp.int32(16))
                        gidx = col0 + v * 16 + l
                        pv = jnp.where(lane == tgt, sc, pv)
                        pi = jnp.where(lane == tgt, gidx, pi)
                        noff = off2 + jnp.where(ok, jnp.int32(1), jnp.int32(0))

                        @pl.when(ok & (lax.rem(noff, 16) == 0))
                        def _():
                            cv_v[pl.ds(noff - 16, 16)] = pv
                            ci_v[pl.ds(noff - 16, 16)] = pi
                        off2 = noff
                    pcv_v[pl.ds(0, 16)] = pv
                    pci_v[pl.ds(0, 16)] = pi
                    return off2

                return lax.cond(_lanemax(val) >= tau, _ins,
                                lambda o: o, off2)

            return lax.fori_loop(0, _CHUNK // 16, _vstep, off2)
        return lax.fori_loop(0, jlim, _chunk, off2)

    n_cand = lax.fori_loop(0, n_batch, _batch, jnp.int32(0))

    @pl.when(lax.rem(n_cand, 16) != 0)
    def _():
        cv_v[pl.ds((n_cand // 16) * 16, 16)] = pcv_v[pl.ds(0, 16)]
        ci_v[pl.ds((n_cand // 16) * 16, 16)] = pci_v[pl.ds(0, 16)]

    nv4 = ((jnp.minimum(n_cand, jnp.int32(_PCAP)) + 15) // 16 + 3) // 4

    def _round(t, acc):
        def _mx(i, m):
            for u in range(4):
                m = jnp.maximum(m, cv_v[pl.ds((i * 4 + u) * 16, 16)])
            return m
        mvec = lax.fori_loop(0, nv4, _mx, jnp.full((16,), _NEG, jnp.float32))
        mx = _lanemax(mvec)

        def _mi(i, mi):
            for u in range(4):
                hit = cv_v[pl.ds((i * 4 + u) * 16, 16)] == mx
                mi = jnp.minimum(mi, jnp.where(
                    hit, ci_v[pl.ds((i * 4 + u) * 16, 16)], jnp.int32(_BIGI)))
            return mi
        ivec = lax.fori_loop(0, nv4, _mi, jnp.full((16,), _BIGI, jnp.int32))
        ix = _lanemin(ivec)

        def _clr(i, c):
            for u in range(4):
                vv = cv_v[pl.ds((i * 4 + u) * 16, 16)]
                cc = ci_v[pl.ds((i * 4 + u) * 16, 16)]
                cv_v[pl.ds((i * 4 + u) * 16, 16)] = jnp.where(
                    vv == mx, jnp.where(cc == ix, jnp.float32(_NEG), vv), vv)
            return c
        lax.fori_loop(0, nv4, _clr, 0)

        acc = jnp.where(lane == lax.rem(t, 16), ix, acc)

        @pl.when(lax.rem(t, 16) == 15)
        def _():
            ti_v[pl.ds((t // 16) * 16, 16)] = acc
        return acc

    lax.fori_loop(0, _K, _round, jnp.full((16,), 0, jnp.int32))

    pltpu.async_copy(v_hbm.at[ti_v], vrows_v, sem).wait()
    pltpu.sync_copy(vrows_v, out_hbm.at[r])


def kernel(query, K_bank, V_bank, topk):
    del topk
    scores, cmax = _phase_a(query, K_bank)
    scores2d = scores.reshape(_B * _NCHUNK, _CHUNK)
    return _phase_b(scores2d, cmax, V_bank)


# speedup vs baseline: 1.0646x; 1.0646x over previous
import functools

import jax
import jax.numpy as jnp
from jax import lax
from jax.experimental import pallas as pl
from jax.experimental.pallas import tpu as pltpu
from jax.experimental.pallas import tpu_sc as plsc

_B, _D, _C, _K = 32, 128, 1000000, 64
_S = 16384
_NBLK = -(-_C // _S)
_CP = _NBLK * _S
_CHUNK = 128
_NCHUNK = _CP // _CHUNK
_NEG = -3.0e38
_BIGI = 0x7FFFFFFF
_GCH = 64
_NFULLG = _C // (_GCH * _CHUNK)
_NCAP = 4096
_PCAP = 4096
_NB = 128
_NC, _NS = 2, 16


def _phase_a_body(q_ref, k_ref, scores_ref, cmax_ref):
    i = pl.program_id(0)
    q = q_ref[...]
    k = k_ref[...]
    s = jax.lax.dot_general(
        q, k, (((1,), (1,)), ((), ())),
        preferred_element_type=jnp.float32)
    col = i * _S + jax.lax.broadcasted_iota(jnp.int32, (_B, _S), 1)
    s = jnp.where(col < _C, s, _NEG)
    s3 = s.reshape(_B, _S // _CHUNK, _CHUNK)
    scores_ref[...] = s3
    cmax_ref[...] = jnp.max(s3, axis=2)


_phase_a = pl.pallas_call(
    _phase_a_body,
    grid=(_NBLK,),
    in_specs=[
        pl.BlockSpec((_B, _D), lambda i: (0, 0)),
        pl.BlockSpec((_S, _D), lambda i: (i, 0)),
    ],
    out_specs=[
        pl.BlockSpec((_B, _S // _CHUNK, _CHUNK), lambda i: (0, i, 0)),
        pl.BlockSpec((_B, _S // _CHUNK), lambda i: (0, i)),
    ],
    out_shape=[
        jax.ShapeDtypeStruct((_B, _NCHUNK, _CHUNK), jnp.float32),
        jax.ShapeDtypeStruct((_B, _NCHUNK), jnp.float32),
    ],
    compiler_params=pltpu.CompilerParams(
        dimension_semantics=("arbitrary",),
    ),
)


@functools.partial(
    pl.kernel,
    mesh=plsc.VectorSubcoreMesh(core_axis_name="c", subcore_axis_name="s"),
    out_type=jax.ShapeDtypeStruct((_B, _K, _D), jnp.float32),
    scratch_types=[
        pltpu.VMEM((_NCHUNK,), jnp.float32),
        pltpu.VMEM((_NCAP + 128,), jnp.int32),
        pltpu.VMEM((_NB, _CHUNK), jnp.float32),
        pltpu.VMEM((_PCAP + 128,), jnp.float32),
        pltpu.VMEM((_PCAP + 128,), jnp.int32),
        pltpu.VMEM((_K,), jnp.int32),
        pltpu.VMEM((_K, _D), jnp.float32),
        pltpu.VMEM((16,), jnp.int32),
        pltpu.VMEM((16,), jnp.float32),
        pltpu.VMEM((16,), jnp.int32),
        pltpu.SemaphoreType.DMA,
    ],
)
def _phase_b(scores2d_hbm, cmax_hbm, v_hbm, out_hbm,
             cmax_v, ids_v, buf_v, cv_v, ci_v, ti_v, vrows_v,
             pend_v, pcv_v, pci_v, sem):
    r = lax.axis_index("s") * _NC + lax.axis_index("c")
    row_base = r * _NCHUNK
    lane = lax.iota(jnp.int32, 16)

    _dnums = lax.GatherDimensionNumbers(
        offset_dims=(), collapsed_slice_dims=(0,), start_index_map=(0,))

    def _lane_shuf(x, k):
        return lax.gather(x, (lane ^ k)[:, None], _dnums, slice_sizes=(1,),
                          mode=lax.GatherScatterMode.PROMISE_IN_BOUNDS)

    def _lanemax(x):
        for k in (8, 4, 2, 1):
            x = jnp.maximum(x, _lane_shuf(x, k))
        return x[0]

    def _lanemin(x):
        for k in (8, 4, 2, 1):
            x = jnp.minimum(x, _lane_shuf(x, k))
        return x[0]

    pltpu.sync_copy(cmax_hbm.at[r], cmax_v)

    def _grp(g, tau):
        m = cmax_v[pl.ds(g * _GCH, 16)]
        for j in range(1, _GCH // 16):
            m = jnp.maximum(m, cmax_v[pl.ds(g * _GCH + j * 16, 16)])
        return jnp.minimum(tau, _lanemax(m))
    tau = lax.fori_loop(0, _NFULLG, _grp, jnp.float32(3.0e38))

    def _fill_ids(i, c):
        ids_v[pl.ds(i * 16, 16)] = row_base + i * 16 + lane
        return c
    lax.fori_loop(0, (_NCAP + 128) // 16, _fill_ids, 0)

    def _fill_cand(i, c):
        cv_v[pl.ds(i * 16, 16)] = jnp.full((16,), _NEG, jnp.float32)
        ci_v[pl.ds(i * 16, 16)] = jnp.full((16,), _BIGI, jnp.int32)
        return c
    lax.fori_loop(0, (_PCAP + 128) // 16, _fill_cand, 0)

    safe_ids = row_base + lane
    pend_v[pl.ds(0, 16)] = safe_ids

    def _sel(i, off):
        def _ins(off):
            v = cmax_v[pl.ds(i * 16, 16)]
            pend = pend_v[pl.ds(0, 16)]
            for l in range(16):
                sc = v[l]
                ok = (sc >= tau) & (off < _NCAP)
                tgt = jnp.where(ok, lax.rem(off, 16), jnp.int32(16))
                pend = jnp.where(lane == tgt, row_base + i * 16 + l, pend)
                noff = off + jnp.where(ok, jnp.int32(1), jnp.int32(0))

                @pl.when(ok & (lax.rem(noff, 16) == 0))
                def _():
                    ids_v[pl.ds(noff - 16, 16)] = pend
                off = noff
            pend_v[pl.ds(0, 16)] = pend
            return off

        v = cmax_v[pl.ds(i * 16, 16)]
        return lax.cond(_lanemax(v) >= tau, _ins, lambda o: o, off)

    n_chunks = lax.fori_loop(0, _NCHUNK // 16, _sel, jnp.int32(0))

    @pl.when(lax.rem(n_chunks, 16) != 0)
    def _():
        ids_v[pl.ds((n_chunks // 16) * 16, 16)] = pend_v[pl.ds(0, 16)]

    n_batch = (n_chunks + _NB - 1) // _NB

    pcv_v[pl.ds(0, 16)] = jnp.full((16,), _NEG, jnp.float32)
    pci_v[pl.ds(0, 16)] = safe_ids

    def _batch(b, off2):
        pltpu.async_copy(
            scores2d_hbm.at[ids_v.at[pl.ds(b * _NB, _NB)]], buf_v, sem,
        ).wait()
        jlim = jnp.minimum(n_chunks - b * _NB, _NB)

        def _chunk(j, off2):
            abs_id = ids_v[pl.ds(b * _NB + j, 16)][0]
            col0 = (abs_id - row_base) * _CHUNK

            def _vstep(v, off2):
                val = buf_v[j, pl.ds(v * 16, 16)]

                def _ins(off2):
                    pv = pcv_v[pl.ds(0, 16)]
                    pi = pci_v[pl.ds(0, 16)]
                    for l in range(16):
                        sc = val[l]
                        ok = (sc >= tau) & (off2 < _PCAP)
                        tgt = jnp.where(ok, lax.rem(off2, 16), jnp.int32(16))
                        gidx = col0 + v * 16 + l
                        pv = jnp.where(lane == tgt, sc, pv)
                        pi = jnp.where(lane == tgt, gidx, pi)
                        noff = off2 + jnp.where(ok, jnp.int32(1), jnp.int32(0))

                        @pl.when(ok & (lax.rem(noff, 16) == 0))
                        def _():
                            cv_v[pl.ds(noff - 16, 16)] = pv
                            ci_v[pl.ds(noff - 16, 16)] = pi
                        off2 = noff
                    pcv_v[pl.ds(0, 16)] = pv
                    pci_v[pl.ds(0, 16)] = pi
                    return off2

                return lax.cond(_lanemax(val) >= tau, _ins,
                                lambda o: o, off2)

            return lax.fori_loop(0, _CHUNK // 16, _vstep, off2)
        return lax.fori_loop(0, jlim, _chunk, off2)

    n_cand = lax.fori_loop(0, n_batch, _batch, jnp.int32(0))

    @pl.when(lax.rem(n_cand, 16) != 0)
    def _():
        cv_v[pl.ds((n_cand // 16) * 16, 16)] = pcv_v[pl.ds(0, 16)]
        ci_v[pl.ds((n_cand // 16) * 16, 16)] = pci_v[pl.ds(0, 16)]

    nv4 = ((jnp.minimum(n_cand, jnp.int32(_PCAP)) + 15) // 16 + 3) // 4

    def _round(t, acc):
        def _mx(i, m):
            for u in range(4):
                m = jnp.maximum(m, cv_v[pl.ds((i * 4 + u) * 16, 16)])
            return m
        mvec = lax.fori_loop(0, nv4, _mx, jnp.full((16,), _NEG, jnp.float32))
        mx = _lanemax(mvec)

        def _mi(i, mi):
            for u in range(4):
                hit = cv_v[pl.ds((i * 4 + u) * 16, 16)] == mx
                mi = jnp.minimum(mi, jnp.where(
                    hit, ci_v[pl.ds((i * 4 + u) * 16, 16)], jnp.int32(_BIGI)))
            return mi
        ivec = lax.fori_loop(0, nv4, _mi, jnp.full((16,), _BIGI, jnp.int32))
        ix = _lanemin(ivec)

        def _clr(i, c):
            for u in range(4):
                vv = cv_v[pl.ds((i * 4 + u) * 16, 16)]
                cc = ci_v[pl.ds((i * 4 + u) * 16, 16)]
                cv_v[pl.ds((i * 4 + u) * 16, 16)] = jnp.where(
                    vv == mx, jnp.where(cc == ix, jnp.float32(_NEG), vv), vv)
            return c
        lax.fori_loop(0, nv4, _clr, 0)

        acc = jnp.where(lane == lax.rem(t, 16), ix, acc)

        @pl.when(lax.rem(t, 16) == 15)
        def _():
            ti_v[pl.ds((t // 16) * 16, 16)] = acc
        return acc

    lax.fori_loop(0, _K, _round, jnp.full((16,), 0, jnp.int32))

    pltpu.async_copy(v_hbm.at[ti_v], vrows_v, sem).wait()
    pltpu.sync_copy(vrows_v, out_hbm.at[r])


def kernel(query, K_bank, V_bank, topk):
    del topk
    scores, cmax = _phase_a(query, K_bank)
    scores2d = scores.reshape(_B * _NCHUNK, _CHUNK)
    return _phase_b(scores2d, cmax, V_bank)
